# baseline (speedup 1.0000x reference)
import jax
import jax.numpy as jnp
import numpy as np
from jax.experimental import pallas as pl

STRIDE = 2
TAU = 1.0
C_BLOCK = 128


def _weights_body(p_ref, g_ref, w_ref, w2_ref):
    z = (p_ref[...] + g_ref[...]) * (1.0 / TAU)
    m = jnp.max(z, axis=1, keepdims=True)
    e = jnp.exp(z - m)
    w = e / jnp.sum(e, axis=1, keepdims=True)
    w_ref[...] = w

    B, _, H, W = w.shape
    lane = jax.lax.broadcasted_iota(jnp.int32, (B, H, 2 * W), 2)
    half = lane // 2
    for i in range(2):
        wa = jnp.take_along_axis(w[:, 2 * i], half, axis=-1)
        wb = jnp.take_along_axis(w[:, 2 * i + 1], half, axis=-1)
        win = jnp.where(lane % 2 == 0, wa, wb)
        w2_ref[:, pl.Slice(i, H, 2), :] = win


def _upsample_body(x_ref, w2_ref, p_ref, o_ref):
    xb = x_ref[0]
    w2 = w2_ref[0]
    pm = p_ref[...]
    cb, hw = xb.shape
    xhi = xb.astype(jnp.bfloat16)
    xlo = (xb - xhi.astype(jnp.float32)).astype(jnp.bfloat16)
    pmb = pm.astype(jnp.bfloat16)
    for k in range(hw // 224):
        hi = jax.lax.dot(
            xhi[:, 224 * k : 224 * (k + 1)], pmb,
            preferred_element_type=jnp.float32,
        )
        lo = jax.lax.dot(
            xlo[:, 224 * k : 224 * (k + 1)], pmb,
            preferred_element_type=jnp.float32,
        )
        o_ref[0, :, 896 * k : 896 * (k + 1)] = (hi + lo) * w2[
            :, 896 * k : 896 * (k + 1)
        ]


def _lps_upsample(x, prob, g):
    B, C, H, W = x.shape
    s = STRIDE

    w, w2 = pl.pallas_call(
        _weights_body,
        out_shape=(
            jax.ShapeDtypeStruct((B, s * s, H, W), jnp.float32),
            jax.ShapeDtypeStruct((B, s * H, s * W), jnp.float32),
        ),
    )(prob, g)

    P = s * s * H * W
    e = np.arange(4 * 224, dtype=np.int64)
    q_np = 56 * (e // 224) + (e % 112) // 2
    pmat = np.zeros((224, 4 * 224), np.float32)
    pmat[q_np, e] = 1.0

    nC = C // C_BLOCK
    out6 = pl.pallas_call(
        _upsample_body,
        grid=(B, nC),
        in_specs=[
            pl.BlockSpec((1, C_BLOCK, H * W), lambda b, c: (b, c, 0)),
            pl.BlockSpec((1, 1, P), lambda b, c: (b, 0, 0)),
            pl.BlockSpec((224, 4 * 224), lambda b, c: (0, 0)),
        ],
        out_specs=pl.BlockSpec((1, C_BLOCK, P), lambda b, c: (b, c, 0)),
        out_shape=jax.ShapeDtypeStruct((B, C, P), jnp.float32),
    )(x.reshape(B, C, H * W), w2.reshape(B, 1, P), jnp.asarray(pmat))
    return out6.reshape(B, C, s * H, s * W), w


def _gumbel(shape):
    gkey = jax.random.key(1234)
    u = jax.random.uniform(gkey, shape, minval=1e-6, maxval=1.0 - 1e-6)
    return -jnp.log(-jnp.log(u))


def kernel(x, prob):
    try:
        with jax.ensure_compile_time_eval():
            g = _gumbel(prob.shape)
    except Exception:
        g = _gumbel(prob.shape)
    return _lps_upsample(x, prob, g)


# speedup vs baseline: 1.0325x; 1.0325x over previous
import jax
import jax.numpy as jnp
import numpy as np
from jax.experimental import pallas as pl

STRIDE = 2
TAU = 1.0
C_BLOCK = 192


def _weights_body(p_ref, g_ref, w_ref, w2_ref):
    z = (p_ref[...] + g_ref[...]) * (1.0 / TAU)
    m = jnp.max(z, axis=1, keepdims=True)
    e = jnp.exp(z - m)
    w = e / jnp.sum(e, axis=1, keepdims=True)
    w_ref[...] = w

    B, _, H, W = w.shape
    lane = jax.lax.broadcasted_iota(jnp.int32, (B, H, 2 * W), 2)
    half = lane // 2
    for i in range(2):
        wa = jnp.take_along_axis(w[:, 2 * i], half, axis=-1)
        wb = jnp.take_along_axis(w[:, 2 * i + 1], half, axis=-1)
        win = jnp.where(lane % 2 == 0, wa, wb)
        w2_ref[:, pl.Slice(i, H, 2), :] = win


def _upsample_body(x_ref, w2_ref, p_ref, o_ref):
    xb = x_ref[0]
    w2 = w2_ref[0]
    pm = p_ref[...]
    cb, hw = xb.shape
    for k in range(hw // 224):
        src = xb[:, 224 * k : 224 * (k + 1)]
        res = jax.lax.dot(src, pm, preferred_element_type=jnp.float32)
        o_ref[0, :, 896 * k : 896 * (k + 1)] = (
            res * w2[:, 896 * k : 896 * (k + 1)]
        )


def _lps_upsample(x, prob, g):
    B, C, H, W = x.shape
    s = STRIDE

    w, w2 = pl.pallas_call(
        _weights_body,
        out_shape=(
            jax.ShapeDtypeStruct((B, s * s, H, W), jnp.float32),
            jax.ShapeDtypeStruct((B, s * H, s * W), jnp.float32),
        ),
    )(prob, g)

    P = s * s * H * W
    e = np.arange(4 * 224, dtype=np.int64)
    q_np = 56 * (e // 224) + (e % 112) // 2
    pmat = np.zeros((224, 4 * 224), np.float32)
    pmat[q_np, e] = 1.0

    nC = C // C_BLOCK
    out6 = pl.pallas_call(
        _upsample_body,
        grid=(B, nC),
        in_specs=[
            pl.BlockSpec((1, C_BLOCK, H * W), lambda b, c: (b, c, 0)),
            pl.BlockSpec((1, 1, P), lambda b, c: (b, 0, 0)),
            pl.BlockSpec((224, 4 * 224), lambda b, c: (0, 0)),
        ],
        out_specs=pl.BlockSpec((1, C_BLOCK, P), lambda b, c: (b, c, 0)),
        out_shape=jax.ShapeDtypeStruct((B, C, P), jnp.float32),
    )(x.reshape(B, C, H * W), w2.reshape(B, 1, P), jnp.asarray(pmat))
    return out6.reshape(B, C, s * H, s * W), w


def _gumbel(shape):
    gkey = jax.random.key(1234)
    u = jax.random.uniform(gkey, shape, minval=1e-6, maxval=1.0 - 1e-6)
    return -jnp.log(-jnp.log(u))


def kernel(x, prob):
    try:
        with jax.ensure_compile_time_eval():
            g = _gumbel(prob.shape)
    except Exception:
        g = _gumbel(prob.shape)
    return _lps_upsample(x, prob, g)


# speedup vs baseline: 1.0375x; 1.0049x over previous
import jax
import jax.numpy as jnp
import numpy as np
from jax.experimental import pallas as pl

STRIDE = 2
TAU = 1.0
C_BLOCK = 384


def _weights_body(p_ref, g_ref, w_ref, w2_ref):
    z = (p_ref[...] + g_ref[...]) * (1.0 / TAU)
    m = jnp.max(z, axis=1, keepdims=True)
    e = jnp.exp(z - m)
    w = e / jnp.sum(e, axis=1, keepdims=True)
    w_ref[...] = w

    B, _, H, W = w.shape
    lane = jax.lax.broadcasted_iota(jnp.int32, (B, H, 2 * W), 2)
    half = lane // 2
    for i in range(2):
        wa = jnp.take_along_axis(w[:, 2 * i], half, axis=-1)
        wb = jnp.take_along_axis(w[:, 2 * i + 1], half, axis=-1)
        win = jnp.where(lane % 2 == 0, wa, wb)
        w2_ref[:, pl.Slice(i, H, 2), :] = win


def _upsample_body(x_ref, w2_ref, p_ref, o_ref):
    xb = x_ref[0]
    w2 = w2_ref[0]
    pm = p_ref[...]
    cb, hw = xb.shape
    for k in range(hw // 224):
        src = xb[:, 224 * k : 224 * (k + 1)]
        res = jax.lax.dot(src, pm, preferred_element_type=jnp.float32)
        o_ref[0, :, 896 * k : 896 * (k + 1)] = (
            res * w2[:, 896 * k : 896 * (k + 1)]
        )


def _lps_upsample(x, prob, g):
    B, C, H, W = x.shape
    s = STRIDE

    w, w2 = pl.pallas_call(
        _weights_body,
        out_shape=(
            jax.ShapeDtypeStruct((B, s * s, H, W), jnp.float32),
            jax.ShapeDtypeStruct((B, s * H, s * W), jnp.float32),
        ),
    )(prob, g)

    P = s * s * H * W
    e = np.arange(4 * 224, dtype=np.int64)
    q_np = 56 * (e // 224) + (e % 112) // 2
    pmat = np.zeros((224, 4 * 224), np.float32)
    pmat[q_np, e] = 1.0

    nC = C // C_BLOCK
    out6 = pl.pallas_call(
        _upsample_body,
        grid=(B, nC),
        in_specs=[
            pl.BlockSpec((1, C_BLOCK, H * W), lambda b, c: (b, c, 0)),
            pl.BlockSpec((1, 1, P), lambda b, c: (b, 0, 0)),
            pl.BlockSpec((224, 4 * 224), lambda b, c: (0, 0)),
        ],
        out_specs=pl.BlockSpec((1, C_BLOCK, P), lambda b, c: (b, c, 0)),
        out_shape=jax.ShapeDtypeStruct((B, C, P), jnp.float32),
    )(x.reshape(B, C, H * W), w2.reshape(B, 1, P), jnp.asarray(pmat))
    return out6.reshape(B, C, s * H, s * W), w


def _gumbel(shape):
    gkey = jax.random.key(1234)
    u = jax.random.uniform(gkey, shape, minval=1e-6, maxval=1.0 - 1e-6)
    return -jnp.log(-jnp.log(u))


def kernel(x, prob):
    try:
        with jax.ensure_compile_time_eval():
            g = _gumbel(prob.shape)
    except Exception:
        g = _gumbel(prob.shape)
    return _lps_upsample(x, prob, g)
